# Initial kernel scaffold; baseline (speedup 1.0000x reference)
import jax
import jax.numpy as jnp
from jax.experimental import pallas as pl


def kernel(x, edge_index, Wl, bl, Wr, br, att, bias, Wout, bout):
    raise NotImplementedError("write your pallas kernel here")


# speedup vs baseline: 14.2417x; 14.2417x over previous
import functools

import jax
import jax.numpy as jnp
from jax import lax
from jax.experimental import pallas as pl
from jax.experimental.pallas import tpu as pltpu
from jax.experimental.pallas import tpu_sc as plsc

NC = 2
NS = 16
NW = NC * NS
CHUNK = 128
ROW = 128


def _gather_body(ew, x_hbm, src_hbm, dst_hbm, xs_out, xd_out, idx_v, rows_v, sem):
    c = lax.axis_index("c")
    s = lax.axis_index("s")
    wid = s * NC + c
    base = wid * ew

    def step(g, carry):
        off = base + g * CHUNK
        pltpu.sync_copy(src_hbm.at[pl.ds(off, CHUNK)], idx_v)
        pltpu.async_copy(x_hbm.at[idx_v], rows_v, sem).wait()
        pltpu.sync_copy(rows_v, xs_out.at[pl.ds(off, CHUNK)])
        pltpu.sync_copy(dst_hbm.at[pl.ds(off, CHUNK)], idx_v)
        pltpu.async_copy(x_hbm.at[idx_v], rows_v, sem).wait()
        pltpu.sync_copy(rows_v, xd_out.at[pl.ds(off, CHUNK)])
        return carry

    lax.fori_loop(0, ew // CHUNK, step, 0)


def _gather_call(xp, src, dst, e_pad):
    ew = e_pad // NW
    mesh = plsc.VectorSubcoreMesh(core_axis_name="c", subcore_axis_name="s")
    f = pl.kernel(
        functools.partial(_gather_body, ew),
        out_type=(
            jax.ShapeDtypeStruct((e_pad, ROW), jnp.float32),
            jax.ShapeDtypeStruct((e_pad, ROW), jnp.float32),
        ),
        mesh=mesh,
        scratch_types=[
            pltpu.VMEM((CHUNK,), jnp.int32),
            pltpu.VMEM((CHUNK, ROW), jnp.float32),
            pltpu.SemaphoreType.DMA,
        ],
    )
    return f(xp, src, dst)


def _logits_body(h, in_dim, xs_ref, xd_ref, w2_ref, b2_ref, att_ref, out_ref):
    xsd = jnp.concatenate(
        [xs_ref[...][:, :in_dim], xd_ref[...][:, :in_dim]], axis=1
    ).astype(jnp.bfloat16)
    rows = []
    for hh in range(h):
        s_h = jnp.dot(xsd, w2_ref[hh], preferred_element_type=jnp.float32)
        s_h = s_h + b2_ref[hh][None, :]
        s_h = jnp.where(s_h > 0, s_h, 0.2 * s_h)
        e_h = jnp.dot(s_h, att_ref[hh], preferred_element_type=jnp.float32)
        rows.append(e_h)
    out_ref[...] = jnp.exp(jnp.stack(rows, axis=0))


def _logits_call(xs, xd, w2, b2, att, e_pad, te):
    h, two_in, c_dim = w2.shape
    grid = (e_pad // te,)
    return pl.pallas_call(
        functools.partial(_logits_body, h, two_in // 2),
        grid=grid,
        in_specs=[
            pl.BlockSpec((te, ROW), lambda i: (i, 0)),
            pl.BlockSpec((te, ROW), lambda i: (i, 0)),
            pl.BlockSpec((h, two_in, c_dim), lambda i: (0, 0, 0)),
            pl.BlockSpec((h, c_dim), lambda i: (0, 0)),
            pl.BlockSpec((h, c_dim), lambda i: (0, 0)),
        ],
        out_specs=pl.BlockSpec((h, te), lambda i: (0, i)),
        out_shape=jax.ShapeDtypeStruct((h, e_pad), jnp.float32),
    )(xs, xd, w2, b2, att)


def _scatter_body(n_acc, e_pad, heads_per_core, in_dim,
                  xs_hbm, expt_hbm, dst_hbm, zeros_hbm, out_hbm,
                  idx_v, xs_v, ex_v, v_buf, accum):
    c = lax.axis_index("c")
    s = lax.axis_index("s")
    rows_per_tile = n_acc // NS
    ew = e_pad // NS
    nchunks = ew // CHUNK
    iota = lax.iota(jnp.int32, 16)
    zero16 = jnp.zeros((16,), jnp.int32)

    pltpu.sync_copy(zeros_hbm.at[pl.ds(0, CHUNK)], v_buf)

    for k in range(heads_per_core):
        hh = c * heads_per_core + k
        pltpu.sync_copy(zeros_hbm.at[pl.ds(s * rows_per_tile, rows_per_tile)],
                        accum.at[pl.ds(s * rows_per_tile, rows_per_tile)])
        plsc.subcore_barrier()

        def chunk_step(ch, carry):
            ebase = s * ew + ch * CHUNK
            pltpu.sync_copy(dst_hbm.at[pl.ds(ebase, CHUNK)], idx_v)
            pltpu.sync_copy(xs_hbm.at[pl.ds(ebase, CHUNK)], xs_v)
            pltpu.sync_copy(expt_hbm.at[pl.ds(hh, 1), pl.ds(ebase, CHUNK)], ex_v)

            def edge_step(j, carry2):
                jv = jnp.full((16,), j, jnp.int32)
                ex = plsc.load_gather(ex_v, [zero16, jv])
                for q in range(in_dim // 16):
                    col = iota + 16 * q
                    xs16 = plsc.load_gather(xs_v, [jv, col])
                    plsc.store_scatter(v_buf, [jv, col], xs16 * ex)
                den = jnp.where(iota == 0, ex, 0.0)
                plsc.store_scatter(v_buf, [jv, iota + in_dim], den)
                return carry2

            lax.fori_loop(0, CHUNK, edge_step, 0)
            pltpu.sync_copy(v_buf, accum.at[idx_v], add=True)
            return carry

        lax.fori_loop(0, nchunks, chunk_step, 0)
        plsc.subcore_barrier()
        pltpu.sync_copy(accum.at[pl.ds(s * rows_per_tile, rows_per_tile)],
                        out_hbm.at[hh, pl.ds(s * rows_per_tile, rows_per_tile)])
        plsc.subcore_barrier()


def _scatter_call(xs, expt, dst_s, zeros_acc, n_acc, e_pad, h, in_dim):
    mesh = plsc.VectorSubcoreMesh(core_axis_name="c", subcore_axis_name="s")
    f = pl.kernel(
        functools.partial(_scatter_body, n_acc, e_pad, h // NC, in_dim),
        out_type=jax.ShapeDtypeStruct((h, n_acc, ROW), jnp.float32),
        mesh=mesh,
        compiler_params=pltpu.CompilerParams(needs_layout_passes=False),
        scratch_types=[
            pltpu.VMEM((CHUNK,), jnp.int32),
            pltpu.VMEM((CHUNK, ROW), jnp.float32),
            pltpu.VMEM((1, CHUNK), jnp.float32),
            pltpu.VMEM((CHUNK, ROW), jnp.float32),
            pltpu.VMEM_SHARED((n_acc, ROW), jnp.float32),
        ],
    )
    return f(xs, expt, dst_s, zeros_acc)


def _head_body(h, in_dim, acc_ref, wl_ref, wout_ref, bv_ref, out_ref):
    tn = acc_ref.shape[1]
    ncls = wout_ref.shape[2]
    acc = jnp.zeros((tn, ncls), jnp.float32)
    for hh in range(h):
        a = (acc_ref[hh, :, 0:in_dim]
             / (acc_ref[hh, :, in_dim:in_dim + 1] + 1e-16))
        emb = jnp.dot(a.astype(jnp.bfloat16), wl_ref[hh],
                      preferred_element_type=jnp.float32) + bv_ref[hh][None, :]
        emb = jnp.maximum(emb, 0.0).astype(jnp.bfloat16)
        acc = acc + jnp.dot(emb, wout_ref[hh], preferred_element_type=jnp.float32)
    out_ref[...] = acc


def _head_call(acc3, wl3, wout3, bv, n_acc, tn):
    h, in_dim, c_dim = wl3.shape
    ncls = wout3.shape[2]
    grid = (n_acc // tn,)
    return pl.pallas_call(
        functools.partial(_head_body, h, in_dim),
        grid=grid,
        in_specs=[
            pl.BlockSpec((h, tn, ROW), lambda i: (0, i, 0)),
            pl.BlockSpec((h, in_dim, c_dim), lambda i: (0, 0, 0)),
            pl.BlockSpec((h, c_dim, ncls), lambda i: (0, 0, 0)),
            pl.BlockSpec((h, c_dim), lambda i: (0, 0)),
        ],
        out_specs=pl.BlockSpec((tn, ncls), lambda i: (i, 0)),
        out_shape=jax.ShapeDtypeStruct((n_acc, ncls), jnp.float32),
    )(acc3, wl3, wout3, bv)


def kernel(x, edge_index, Wl, bl, Wr, br, att, bias, Wout, bout):
    n, in_dim = x.shape
    e = edge_index.shape[1]
    h, c_dim = att.shape
    ncls = Wout.shape[1]

    et = e + n
    e_pad = (NW * CHUNK) * -(-et // (NW * CHUNK))
    tn = 256
    n_acc = tn * -(-(n + 64) // tn)
    npad = e_pad - et

    loop = jnp.arange(n, dtype=edge_index.dtype)
    padfill = jnp.arange(npad, dtype=edge_index.dtype) % n
    src = jnp.concatenate([edge_index[0], loop, padfill])
    dst_g = jnp.concatenate([edge_index[1], loop, padfill])
    dst_s = jnp.concatenate(
        [edge_index[1], loop,
         n + (jnp.arange(npad, dtype=edge_index.dtype) % (n_acc - n))])

    xp = jnp.pad(x, ((0, 0), (0, ROW - in_dim)))
    xs, xd = _gather_call(xp, src, dst_g, e_pad)

    w2 = jnp.concatenate([Wl, Wr], axis=0).reshape(2 * in_dim, h, c_dim)
    w2 = jnp.transpose(w2, (1, 0, 2)).astype(jnp.bfloat16)
    b2 = (bl + br).reshape(h, c_dim)
    expt = _logits_call(xs, xd, w2, b2, att, e_pad, te=1024)

    zeros_acc = jnp.zeros((n_acc, ROW), jnp.float32)
    acc3 = _scatter_call(xs, expt, dst_s, zeros_acc, n_acc, e_pad, h, in_dim)

    wl3 = jnp.transpose(Wl.reshape(in_dim, h, c_dim), (1, 0, 2)).astype(jnp.bfloat16)
    wout3 = Wout.reshape(h, c_dim, ncls).astype(jnp.bfloat16)
    bv = bl.reshape(h, c_dim) + bias.reshape(h, c_dim)
    out = _head_call(acc3, wl3, wout3, bv, n_acc, tn)

    return out[:n] + bout[None, :]


# speedup vs baseline: 26.8950x; 1.8885x over previous
import functools

import jax
import jax.numpy as jnp
from jax import lax
from jax.experimental import pallas as pl
from jax.experimental.pallas import tpu as pltpu
from jax.experimental.pallas import tpu_sc as plsc

NC = 2
NS = 16
NW = NC * NS
CHUNK = 128
CH3 = 64
ROW = 128


def _gather_body(ew, x_hbm, src_hbm, dst_hbm, xs_out, xd_out, idx_v, rows_v, sem):
    c = lax.axis_index("c")
    s = lax.axis_index("s")
    wid = s * NC + c
    base = wid * ew

    def step(g, carry):
        off = base + g * CHUNK
        pltpu.sync_copy(src_hbm.at[pl.ds(off, CHUNK)], idx_v)
        pltpu.async_copy(x_hbm.at[idx_v], rows_v, sem).wait()
        pltpu.sync_copy(rows_v, xs_out.at[pl.ds(off, CHUNK)])
        pltpu.sync_copy(dst_hbm.at[pl.ds(off, CHUNK)], idx_v)
        pltpu.async_copy(x_hbm.at[idx_v], rows_v, sem).wait()
        pltpu.sync_copy(rows_v, xd_out.at[pl.ds(off, CHUNK)])
        return carry

    lax.fori_loop(0, ew // CHUNK, step, 0)


def _gather_call(xp, src, dst, e_pad):
    ew = e_pad // NW
    mesh = plsc.VectorSubcoreMesh(core_axis_name="c", subcore_axis_name="s")
    f = pl.kernel(
        functools.partial(_gather_body, ew),
        out_type=(
            jax.ShapeDtypeStruct((e_pad, ROW), jnp.float32),
            jax.ShapeDtypeStruct((e_pad, ROW), jnp.float32),
        ),
        mesh=mesh,
        scratch_types=[
            pltpu.VMEM((CHUNK,), jnp.int32),
            pltpu.VMEM((CHUNK, ROW), jnp.float32),
            pltpu.SemaphoreType.DMA,
        ],
    )
    return f(xp, src, dst)


def _logits_body(h, in_dim, xs_ref, xd_ref, w2_ref, b2_ref, att_ref, out_ref):
    xsd = jnp.concatenate(
        [xs_ref[...][:, :in_dim], xd_ref[...][:, :in_dim]], axis=1
    ).astype(jnp.bfloat16)
    rows = []
    for hh in range(h):
        s_h = jnp.dot(xsd, w2_ref[hh], preferred_element_type=jnp.float32)
        s_h = s_h + b2_ref[hh][None, :]
        s_h = jnp.maximum(s_h, 0.2 * s_h)
        e_h = jnp.dot(s_h, att_ref[hh], preferred_element_type=jnp.float32)
        rows.append(e_h)
    out_ref[...] = jnp.exp(jnp.stack(rows, axis=0))


def _logits_call(xs, xd, w2, b2, att, e_pad, te):
    h, two_in, c_dim = w2.shape
    grid = (e_pad // te,)
    return pl.pallas_call(
        functools.partial(_logits_body, h, two_in // 2),
        grid=grid,
        in_specs=[
            pl.BlockSpec((te, ROW), lambda i: (i, 0)),
            pl.BlockSpec((te, ROW), lambda i: (i, 0)),
            pl.BlockSpec((h, two_in, c_dim), lambda i: (0, 0, 0)),
            pl.BlockSpec((h, c_dim), lambda i: (0, 0)),
            pl.BlockSpec((h, c_dim), lambda i: (0, 0)),
        ],
        out_specs=pl.BlockSpec((h, te), lambda i: (0, i)),
        out_shape=jax.ShapeDtypeStruct((h, e_pad), jnp.float32),
    )(xs, xd, w2, b2, att)


def _scatter_body(n_acc, e_pad, heads_per_core, in_dim,
                  xs_hbm, expt_hbm, dst_hbm, zeros_hbm, out_hbm,
                  idx0, idx1, xs0, xs1, ex0, ex1, vb0, vb1, accum,
                  sin0, sin1, ssc0, ssc1):
    c = lax.axis_index("c")
    s = lax.axis_index("s")
    rows_per_tile = n_acc // NS
    ew = e_pad // NS
    nchunks = ew // CH3
    iota = lax.iota(jnp.int32, 16)
    zero16 = jnp.zeros((16,), jnp.int32)
    idx = (idx0, idx1)
    xsb = (xs0, xs1)
    exb = (ex0, ex1)
    vb = (vb0, vb1)
    sin = (sin0, sin1)
    ssc = (ssc0, ssc1)

    def start_in(hh, g, b):
        ebase = s * ew + g * CH3
        pltpu.async_copy(dst_hbm.at[pl.ds(ebase, CH3)], idx[b], sin[b])
        pltpu.async_copy(xs_hbm.at[pl.ds(ebase, CH3)], xsb[b], sin[b])
        pltpu.async_copy(expt_hbm.at[pl.ds(hh * e_pad + ebase, CH3)],
                         exb[b], sin[b])

    def wait_in(b):
        pltpu.make_async_copy(dst_hbm.at[pl.ds(0, CH3)], idx[b], sin[b]).wait()
        pltpu.make_async_copy(xs_hbm.at[pl.ds(0, CH3)], xsb[b], sin[b]).wait()
        pltpu.make_async_copy(expt_hbm.at[pl.ds(0, CH3)], exb[b], sin[b]).wait()

    def compute(b):
        xs_v, ex_v, v_buf = xsb[b], exb[b], vb[b]

        @plsc.parallel_loop(0, CH3, 1, unroll=4)
        def edge_step(j):
            jv = jnp.full((16,), j, jnp.int32)
            ex = plsc.load_gather(ex_v, [jv])
            for q in range(in_dim // 16):
                col = iota + 16 * q
                xs16 = plsc.load_gather(xs_v, [jv, col])
                plsc.store_scatter(v_buf, [jv, col], xs16 * ex)
            den = jnp.where(iota == 0, ex, 0.0)
            plsc.store_scatter(v_buf, [jv, iota + in_dim], den)

    def start_sc(b):
        pltpu.async_copy(vb[b], accum.at[idx[b]], ssc[b], add=True)

    def wait_sc(b):
        pltpu.make_async_copy(vb[b], accum.at[idx[b]], ssc[b]).wait()

    pltpu.sync_copy(zeros_hbm.at[pl.ds(0, CH3)], vb0)
    pltpu.sync_copy(zeros_hbm.at[pl.ds(0, CH3)], vb1)

    for k in range(heads_per_core):
        hh = c * heads_per_core + k
        pltpu.sync_copy(zeros_hbm.at[pl.ds(s * rows_per_tile, rows_per_tile)],
                        accum.at[pl.ds(s * rows_per_tile, rows_per_tile)])
        plsc.subcore_barrier()

        start_in(hh, 0, 0)
        start_in(hh, 1, 1)
        for b in (0, 1):
            wait_in(b)
            compute(b)
            start_sc(b)
            start_in(hh, 2 + b, b)

        def pair_step(g2, carry):
            for b in (0, 1):
                g = 2 * g2 + b
                wait_in(b)
                wait_sc(b)
                compute(b)
                start_sc(b)

                @pl.when(g + 2 < nchunks)
                def _():
                    start_in(hh, g + 2, b)
            return carry

        lax.fori_loop(1, nchunks // 2, pair_step, 0)
        wait_sc(0)
        wait_sc(1)
        plsc.subcore_barrier()
        pltpu.sync_copy(accum.at[pl.ds(s * rows_per_tile, rows_per_tile)],
                        out_hbm.at[hh, pl.ds(s * rows_per_tile, rows_per_tile)])
        plsc.subcore_barrier()


def _scatter_call(xs, expt, dst_s, zeros_acc, n_acc, e_pad, h, in_dim):
    mesh = plsc.VectorSubcoreMesh(core_axis_name="c", subcore_axis_name="s")
    f = pl.kernel(
        functools.partial(_scatter_body, n_acc, e_pad, h // NC, in_dim),
        out_type=jax.ShapeDtypeStruct((h, n_acc, ROW), jnp.float32),
        mesh=mesh,
        compiler_params=pltpu.CompilerParams(needs_layout_passes=False),
        scratch_types=[
            pltpu.VMEM((CH3,), jnp.int32),
            pltpu.VMEM((CH3,), jnp.int32),
            pltpu.VMEM((CH3, ROW), jnp.float32),
            pltpu.VMEM((CH3, ROW), jnp.float32),
            pltpu.VMEM((CH3,), jnp.float32),
            pltpu.VMEM((CH3,), jnp.float32),
            pltpu.VMEM((CH3, ROW), jnp.float32),
            pltpu.VMEM((CH3, ROW), jnp.float32),
            pltpu.VMEM_SHARED((n_acc, ROW), jnp.float32),
            pltpu.SemaphoreType.DMA,
            pltpu.SemaphoreType.DMA,
            pltpu.SemaphoreType.DMA,
            pltpu.SemaphoreType.DMA,
        ],
    )
    return f(xs, expt.reshape(-1), dst_s, zeros_acc)


def _head_body(h, in_dim, acc_ref, wl_ref, wout_ref, bv_ref, out_ref):
    tn = acc_ref.shape[1]
    ncls = wout_ref.shape[2]
    acc = jnp.zeros((tn, ncls), jnp.float32)
    for hh in range(h):
        a = (acc_ref[hh, :, 0:in_dim]
             / (acc_ref[hh, :, in_dim:in_dim + 1] + 1e-16))
        emb = jnp.dot(a.astype(jnp.bfloat16), wl_ref[hh],
                      preferred_element_type=jnp.float32) + bv_ref[hh][None, :]
        emb = jnp.maximum(emb, 0.0).astype(jnp.bfloat16)
        acc = acc + jnp.dot(emb, wout_ref[hh], preferred_element_type=jnp.float32)
    out_ref[...] = acc


def _head_call(acc3, wl3, wout3, bv, n_acc, tn):
    h, in_dim, c_dim = wl3.shape
    ncls = wout3.shape[2]
    grid = (n_acc // tn,)
    return pl.pallas_call(
        functools.partial(_head_body, h, in_dim),
        grid=grid,
        in_specs=[
            pl.BlockSpec((h, tn, ROW), lambda i: (0, i, 0)),
            pl.BlockSpec((h, in_dim, c_dim), lambda i: (0, 0, 0)),
            pl.BlockSpec((h, c_dim, ncls), lambda i: (0, 0, 0)),
            pl.BlockSpec((h, c_dim), lambda i: (0, 0)),
        ],
        out_specs=pl.BlockSpec((tn, ncls), lambda i: (i, 0)),
        out_shape=jax.ShapeDtypeStruct((n_acc, ncls), jnp.float32),
    )(acc3, wl3, wout3, bv)


def kernel(x, edge_index, Wl, bl, Wr, br, att, bias, Wout, bout):
    n, in_dim = x.shape
    e = edge_index.shape[1]
    h, c_dim = att.shape
    ncls = Wout.shape[1]

    et = e + n
    e_pad = (NW * CHUNK) * -(-et // (NW * CHUNK))
    tn = 256
    n_acc = tn * -(-(n + 64) // tn)
    npad = e_pad - et

    loop = jnp.arange(n, dtype=edge_index.dtype)
    padfill = jnp.arange(npad, dtype=edge_index.dtype) % n
    src = jnp.concatenate([edge_index[0], loop, padfill])
    dst_g = jnp.concatenate([edge_index[1], loop, padfill])
    dst_s = jnp.concatenate(
        [edge_index[1], loop,
         n + (jnp.arange(npad, dtype=edge_index.dtype) % (n_acc - n))])

    xp = jnp.pad(x, ((0, 0), (0, ROW - in_dim)))
    xs, xd = _gather_call(xp, src, dst_g, e_pad)

    w2 = jnp.concatenate([Wl, Wr], axis=0).reshape(2 * in_dim, h, c_dim)
    w2 = jnp.transpose(w2, (1, 0, 2)).astype(jnp.bfloat16)
    b2 = (bl + br).reshape(h, c_dim)
    expt = _logits_call(xs, xd, w2, b2, att, e_pad, te=1024)

    zeros_acc = jnp.zeros((n_acc, ROW), jnp.float32)
    acc3 = _scatter_call(xs, expt, dst_s, zeros_acc, n_acc, e_pad, h, in_dim)

    wl3 = jnp.transpose(Wl.reshape(in_dim, h, c_dim), (1, 0, 2)).astype(jnp.bfloat16)
    wout3 = Wout.reshape(h, c_dim, ncls).astype(jnp.bfloat16)
    bv = bl.reshape(h, c_dim) + bias.reshape(h, c_dim)
    out = _head_call(acc3, wl3, wout3, bv, n_acc, tn)

    return out[:n] + bout[None, :]
